# baseline (speedup 1.0000x reference)
import functools

import jax
import jax.numpy as jnp
from jax import lax
from jax.experimental import pallas as pl
from jax.experimental.pallas import tpu as pltpu
from jax.experimental.pallas import tpu_sc as plsc

_D = 128
_N = 100000
_NC, _NS = 2, 16
_NW = _NC * _NS
_C = 448
_CHUNKS = 7
_ROWS_W = _C * _CHUNKS
_LAST_BASE = _N - _ROWS_W

_mesh = plsc.VectorSubcoreMesh(core_axis_name="c", subcore_axis_name="s")


@functools.partial(
    pl.kernel,
    out_type=jax.ShapeDtypeStruct((_N, _D), jnp.float32),
    mesh=_mesh,
    scratch_types=[
        pltpu.VMEM((_C,), jnp.int32),
        pltpu.VMEM((_C,), jnp.int32),
        pltpu.VMEM((_C, _D), jnp.float32),
        pltpu.VMEM((_C, _D), jnp.float32),
        pltpu.SemaphoreType.DMA,
        pltpu.SemaphoreType.DMA,
    ],
)
def _embed_gather(types_hbm, table_hbm, out_hbm,
                  idx0, idx1, rows0, rows1, sem0, sem1):
    w = lax.axis_index("s") * _NC + lax.axis_index("c")
    base = ((w * _LAST_BASE) // (_NW - 1)) // 8 * 8

    idx = (idx0, idx1)
    rows = (rows0, rows1)
    sems = (sem0, sem1)
    handles = [None, None]

    pltpu.sync_copy(types_hbm.at[pl.ds(base, _C)], idx[0])
    handles[0] = pltpu.make_async_copy(table_hbm.at[idx[0]], rows[0], sems[0])
    handles[0].start()

    for g in range(_CHUNKS):
        b = g % 2
        if g + 1 < _CHUNKS:
            nb = (g + 1) % 2
            pltpu.sync_copy(
                types_hbm.at[pl.ds(base + (g + 1) * _C, _C)], idx[nb])
            handles[nb] = pltpu.make_async_copy(
                table_hbm.at[idx[nb]], rows[nb], sems[nb])
            handles[nb].start()
        handles[b].wait()
        pltpu.sync_copy(rows[b], out_hbm.at[pl.ds(base + g * _C, _C)])


def kernel(atom_types, embed_table):
    flat_types = atom_types.reshape(-1).astype(jnp.int32)
    out = _embed_gather(flat_types, embed_table)
    return (out, out)


# speedup vs baseline: 2.9005x; 2.9005x over previous
import functools

import jax
import jax.numpy as jnp
from jax import lax
from jax.experimental import pallas as pl
from jax.experimental.pallas import tpu as pltpu
from jax.experimental.pallas import tpu_sc as plsc

_D = 128
_N = 100000
_NC, _NS = 2, 16
_NW = _NC * _NS
_C = 448
_CHUNKS = 7
_ROWS_W = _C * _CHUNKS
_LAST_BASE = _N - _ROWS_W

_mesh = plsc.VectorSubcoreMesh(core_axis_name="c", subcore_axis_name="s")


@functools.partial(
    pl.kernel,
    out_type=jax.ShapeDtypeStruct((_N, _D), jnp.float32),
    mesh=_mesh,
    scratch_types=[
        pltpu.VMEM((_C,), jnp.int32),
        pltpu.VMEM((_C,), jnp.int32),
        pltpu.VMEM((_C, _D), jnp.float32),
        pltpu.VMEM((_C, _D), jnp.float32),
        pltpu.SemaphoreType.DMA,
        pltpu.SemaphoreType.DMA,
    ],
)
def _embed_gather(types_hbm, table_hbm, out_hbm,
                  idx0, idx1, rows0, rows1, sem0, sem1):
    w = lax.axis_index("s") * _NC + lax.axis_index("c")
    base = ((w * _LAST_BASE) // (_NW - 1)) // 8 * 8

    idx = (idx0, idx1)
    rows = (rows0, rows1)
    sems = (sem0, sem1)
    handles = [None, None]

    shift = w * 16

    def _stage(g, buf):
        pltpu.sync_copy(types_hbm.at[pl.ds(base + g * _C, _C)], idx[buf])
        for k in range(_C // 16):
            sl = pl.ds(k * 16, 16)
            idx[buf][sl] = idx[buf][sl] + shift
        handles[buf] = pltpu.make_async_copy(
            table_hbm.at[idx[buf]], rows[buf], sems[buf])
        handles[buf].start()

    _stage(0, 0)

    for g in range(_CHUNKS):
        b = g % 2
        if g + 1 < _CHUNKS:
            _stage(g + 1, (g + 1) % 2)
        handles[b].wait()
        pltpu.sync_copy(rows[b], out_hbm.at[pl.ds(base + g * _C, _C)])


def kernel(atom_types, embed_table):
    flat_types = atom_types.reshape(-1).astype(jnp.int32)
    table_rep = jnp.tile(embed_table, (_NW, 1))
    out = _embed_gather(flat_types, table_rep)
    return (out, out)


# speedup vs baseline: 3.6713x; 1.2657x over previous
import functools

import jax
import jax.numpy as jnp
from jax import lax
from jax.experimental import pallas as pl
from jax.experimental.pallas import tpu as pltpu
from jax.experimental.pallas import tpu_sc as plsc

_D = 128
_N = 100000
_NC, _NS = 2, 16
_NW = _NC * _NS
_C = 448
_CHUNKS = 7
_ROWS_W = _C * _CHUNKS
_LAST_BASE = _N - _ROWS_W
_REP = 4

_mesh = plsc.VectorSubcoreMesh(core_axis_name="c", subcore_axis_name="s")


@functools.partial(
    pl.kernel,
    out_type=jax.ShapeDtypeStruct((_N, _D), jnp.float32),
    mesh=_mesh,
    scratch_types=[
        pltpu.VMEM((_C,), jnp.int32),
        pltpu.VMEM((_C,), jnp.int32),
        pltpu.VMEM((_C, _D), jnp.float32),
        pltpu.VMEM((_C, _D), jnp.float32),
        pltpu.SemaphoreType.DMA,
        pltpu.SemaphoreType.DMA,
    ],
)
def _embed_gather(types_hbm, table_hbm, out_hbm,
                  idx0, idx1, rows0, rows1, sem0, sem1):
    w = lax.axis_index("s") * _NC + lax.axis_index("c")
    base = ((w * _LAST_BASE) // (_NW - 1)) // 8 * 8

    idx = (idx0, idx1)
    rows = (rows0, rows1)
    sems = (sem0, sem1)
    handles = [None, None]

    lane = lax.broadcasted_iota(jnp.int32, (16,), 0)

    def _stage(g, buf):
        pltpu.sync_copy(types_hbm.at[pl.ds(base + g * _C, _C)], idx[buf])
        for k in range(_C // 16):
            sl = pl.ds(k * 16, 16)
            rep = w * _REP + (lane + k) % _REP
            idx[buf][sl] = idx[buf][sl] + rep * 16
        handles[buf] = pltpu.make_async_copy(
            table_hbm.at[idx[buf]], rows[buf], sems[buf])
        handles[buf].start()

    _stage(0, 0)

    for g in range(_CHUNKS):
        b = g % 2
        if g + 1 < _CHUNKS:
            _stage(g + 1, (g + 1) % 2)
        handles[b].wait()
        pltpu.sync_copy(rows[b], out_hbm.at[pl.ds(base + g * _C, _C)])


def kernel(atom_types, embed_table):
    flat_types = atom_types.reshape(-1).astype(jnp.int32)
    table_rep = jnp.tile(embed_table, (_NW * _REP, 1))
    out = _embed_gather(flat_types, table_rep)
    return (out, out)


# speedup vs baseline: 3.9853x; 1.0855x over previous
import functools

import jax
import jax.numpy as jnp
from jax import lax
from jax.experimental import pallas as pl
from jax.experimental.pallas import tpu as pltpu
from jax.experimental.pallas import tpu_sc as plsc

_D = 128
_N = 100000
_NC, _NS = 2, 16
_NW = _NC * _NS
_C = 448
_CHUNKS = 7
_ROWS_W = _C * _CHUNKS
_LAST_BASE = _N - _ROWS_W
_REP = 8

_mesh = plsc.VectorSubcoreMesh(core_axis_name="c", subcore_axis_name="s")


@functools.partial(
    pl.kernel,
    out_type=jax.ShapeDtypeStruct((_N, _D), jnp.float32),
    mesh=_mesh,
    scratch_types=[
        pltpu.VMEM((_C,), jnp.int32),
        pltpu.VMEM((_C,), jnp.int32),
        pltpu.VMEM((_C, _D), jnp.float32),
        pltpu.VMEM((_C, _D), jnp.float32),
        pltpu.SemaphoreType.DMA,
        pltpu.SemaphoreType.DMA,
        pltpu.SemaphoreType.DMA,
        pltpu.SemaphoreType.DMA,
    ],
)
def _embed_gather(types_hbm, table_hbm, out_hbm,
                  idx0, idx1, rows0, rows1, sem0, sem1, wsem0, wsem1):
    w = lax.axis_index("s") * _NC + lax.axis_index("c")
    base = ((w * _LAST_BASE) // (_NW - 1)) // 8 * 8

    idx = (idx0, idx1)
    rows = (rows0, rows1)
    sems = (sem0, sem1)
    wsems = (wsem0, wsem1)
    handles = [None, None]
    whandles = [None, None]

    lane = lax.broadcasted_iota(jnp.int32, (16,), 0)

    def _stage(g, buf):
        pltpu.sync_copy(types_hbm.at[pl.ds(base + g * _C, _C)], idx[buf])
        for k in range(_C // 16):
            sl = pl.ds(k * 16, 16)
            rep = w * _REP + (lane + k) % _REP
            idx[buf][sl] = idx[buf][sl] + rep * 16
        if whandles[buf] is not None:
            whandles[buf].wait()
        handles[buf] = pltpu.make_async_copy(
            table_hbm.at[idx[buf]], rows[buf], sems[buf])
        handles[buf].start()

    _stage(0, 0)

    for g in range(_CHUNKS):
        b = g % 2
        if g + 1 < _CHUNKS:
            _stage(g + 1, (g + 1) % 2)
        handles[b].wait()
        whandles[b] = pltpu.make_async_copy(
            rows[b], out_hbm.at[pl.ds(base + g * _C, _C)], wsems[b])
        whandles[b].start()

    whandles[(_CHUNKS - 1) % 2].wait()
    whandles[_CHUNKS % 2].wait()


def kernel(atom_types, embed_table):
    flat_types = atom_types.reshape(-1).astype(jnp.int32)
    table_rep = jnp.tile(embed_table, (_NW * _REP, 1))
    out = _embed_gather(flat_types, table_rep)
    return (out, out)


# speedup vs baseline: 5.0137x; 1.2580x over previous
import functools

import jax
import jax.numpy as jnp
from jax import lax
from jax.experimental import pallas as pl
from jax.experimental.pallas import tpu as pltpu
from jax.experimental.pallas import tpu_sc as plsc

_D = 128
_N = 100000
_NC, _NS = 2, 16
_NW = _NC * _NS
_C = 448
_CHUNKS = 7
_ROWS_W = _C * _CHUNKS
_LAST_BASE = _N - _ROWS_W

_mesh = plsc.VectorSubcoreMesh(core_axis_name="c", subcore_axis_name="s")


@functools.partial(
    pl.kernel,
    out_type=jax.ShapeDtypeStruct((_N, _D), jnp.float32),
    mesh=_mesh,
    scratch_types=[
        pltpu.VMEM_SHARED((16, _D), jnp.float32),
        pltpu.VMEM((_C,), jnp.int32),
        pltpu.VMEM((_C,), jnp.int32),
        pltpu.VMEM((_C, _D), jnp.float32),
        pltpu.VMEM((_C, _D), jnp.float32),
        pltpu.SemaphoreType.DMA,
        pltpu.SemaphoreType.DMA,
        pltpu.SemaphoreType.DMA,
        pltpu.SemaphoreType.DMA,
    ],
)
def _embed_gather(types_hbm, table_hbm, out_hbm,
                  table_v, idx0, idx1, rows0, rows1,
                  sem0, sem1, wsem0, wsem1):
    w = lax.axis_index("s") * _NC + lax.axis_index("c")
    base = ((w * _LAST_BASE) // (_NW - 1)) // 8 * 8

    idx = (idx0, idx1)
    rows = (rows0, rows1)
    sems = (sem0, sem1)
    wsems = (wsem0, wsem1)
    handles = [None, None]
    whandles = [None, None]

    @pl.when(lax.axis_index("s") == 0)
    def _():
        pltpu.sync_copy(table_hbm, table_v)

    plsc.subcore_barrier()

    def _stage(g, buf):
        pltpu.sync_copy(types_hbm.at[pl.ds(base + g * _C, _C)], idx[buf])
        if whandles[buf] is not None:
            whandles[buf].wait()
        handles[buf] = pltpu.make_async_copy(
            table_v.at[idx[buf]], rows[buf], sems[buf])
        handles[buf].start()

    _stage(0, 0)

    for g in range(_CHUNKS):
        b = g % 2
        if g + 1 < _CHUNKS:
            _stage(g + 1, (g + 1) % 2)
        handles[b].wait()
        whandles[b] = pltpu.make_async_copy(
            rows[b], out_hbm.at[pl.ds(base + g * _C, _C)], wsems[b])
        whandles[b].start()

    whandles[(_CHUNKS - 1) % 2].wait()
    whandles[_CHUNKS % 2].wait()


def kernel(atom_types, embed_table):
    flat_types = atom_types.reshape(-1).astype(jnp.int32)
    out = _embed_gather(flat_types, embed_table)
    return (out, out)
